# baseline (speedup 1.0000x reference)
import functools

import jax
import jax.numpy as jnp
from jax.experimental import pallas as pl
from jax.experimental.pallas import tpu as pltpu
from jax.experimental.pallas import tpu_sc as plsc

BATCH = 4096
MAX_LEN = 200
EMBED_DIM = 32
NUM_IDX = BATCH * MAX_LEN
WINDOW = 128
GATHERS_PER_BODY = 8
BLOCK = WINDOW * GATHERS_PER_BODY

T4 = MAX_LEN // 4
BB = BATCH // 128


VOCAB = 1000000
VP = 1 << 20
GS = VP // 4
NB = GS // 128
LAST_BLK = (VOCAB - 1) // 128


def _table_relayout_tc(tt):
    def body(t0, t1, t2, t3, o_ref):
        stacked = jnp.concatenate(
            [t0[...], t1[...], t2[...], t3[...]], axis=0
        )
        o_ref[...] = stacked.T

    def imap(p):
        return lambda i: (0, jnp.minimum(p * NB + i, LAST_BLK))

    return pl.pallas_call(
        body,
        grid=(NB,),
        in_specs=[pl.BlockSpec((32, 128), imap(p)) for p in range(4)],
        out_specs=pl.BlockSpec((128, 128), lambda i: (i, 0)),
        out_shape=jax.ShapeDtypeStruct((GS, 128), jnp.float32),
    )(tt, tt, tt, tt)


def _gather_sc(table, idx_flat):
    mesh = plsc.VectorSubcoreMesh(core_axis_name="c", subcore_axis_name="s")

    @functools.partial(
        pl.kernel,
        out_type=jax.ShapeDtypeStruct((NUM_IDX, EMBED_DIM), table.dtype),
        mesh=mesh,
        scratch_types=[pltpu.SemaphoreType.DMA],
        compiler_params=pltpu.CompilerParams(use_tc_tiling_on_sc=False),
    )
    def gather_kernel(table_hbm, idx_hbm, out_hbm, sem):
        def body(i_vmem, o_vmem):
            copies = [
                pltpu.async_copy(
                    table_hbm.at[i_vmem.at[0, pl.ds(k * WINDOW, WINDOW)]],
                    o_vmem.at[pl.ds(k * WINDOW, WINDOW)],
                    sem,
                )
                for k in range(GATHERS_PER_BODY)
            ]
            for c in copies:
                c.wait()

        pltpu.emit_pipeline(
            body,
            grid=(NUM_IDX // BLOCK,),
            in_specs=[
                pl.BlockSpec((1, BLOCK), index_map=lambda i: (0, i)),
            ],
            out_specs=[
                pl.BlockSpec((BLOCK, EMBED_DIM), index_map=lambda i: (i, 0)),
            ],
            core_axis_name=("c", "s"),
            dimension_semantics=(pltpu.PARALLEL,),
        )(idx_hbm, out_hbm)

    return gather_kernel(table, idx_flat)


def _relayout_tc(x128):
    def body(x_ref, o_ref):
        x = x_ref[...]
        x3 = x.reshape(128, T4, 128)
        for t4 in range(T4):
            y = x3[:, t4, :].T
            o_ref[pl.ds(4 * t4, 4), :, 0, :, :] = y.reshape(4, 4, 8, 128)

    return pl.pallas_call(
        body,
        grid=(BB,),
        in_specs=[pl.BlockSpec((128 * T4, 128), lambda i: (i, 0))],
        out_specs=pl.BlockSpec(
            (MAX_LEN, 4, 1, 8, 128), lambda i: (0, 0, i, 0, 0)
        ),
        out_shape=jax.ShapeDtypeStruct(
            (MAX_LEN, 4, BB, 8, 128), jnp.float32
        ),
    )(x128)


def kernel(inputs, table):
    idx = inputs.reshape(1, NUM_IDX).astype(jnp.int32)
    idx_flat = (idx % GS) * 4 + idx // GS
    tt = jnp.swapaxes(table, 0, 1)
    table4 = _table_relayout_tc(tt).reshape(VP, 32)
    rows = _gather_sc(table4, idx_flat)
    d = _relayout_tc(rows.reshape(NUM_IDX // 4, 128))
    return d.transpose(2, 4, 0, 1, 3).reshape(BATCH, MAX_LEN, EMBED_DIM)


# speedup vs baseline: 3.6644x; 3.6644x over previous
import functools

import jax
import jax.numpy as jnp
from jax.experimental import pallas as pl
from jax.experimental.pallas import tpu as pltpu
from jax.experimental.pallas import tpu_sc as plsc

BATCH = 4096
MAX_LEN = 200
EMBED_DIM = 32
NUM_IDX = BATCH * MAX_LEN
WINDOW = 128
GATHERS_PER_BODY = 8
BLOCK = WINDOW * GATHERS_PER_BODY

T4 = MAX_LEN // 4
BB = BATCH // 128


VOCAB = 1000000
VP = 1 << 20
GS = VP // 4
NB = GS // 128
LAST_BLK = (VOCAB - 1) // 128


def _table_relayout_tc(tt):
    def body(t0, t1, t2, t3, o_ref):
        xs = [t0[...], t1[...], t2[...], t3[...]]
        for g in range(16):
            stacked = jnp.concatenate(
                [x[:, 128 * g : 128 * (g + 1)] for x in xs], axis=0
            )
            o_ref[pl.ds(128 * g, 128), :] = stacked.T

    nb2 = GS // 2048
    last2 = (VOCAB - 1) // 2048

    def imap(p):
        return lambda i: (0, jnp.minimum(p * nb2 + i, last2))

    return pl.pallas_call(
        body,
        grid=(nb2,),
        in_specs=[pl.BlockSpec((32, 2048), imap(p)) for p in range(4)],
        out_specs=pl.BlockSpec((2048, 128), lambda i: (i, 0)),
        out_shape=jax.ShapeDtypeStruct((GS, 128), jnp.float32),
    )(tt, tt, tt, tt)


def _gather_sc(table, idx_flat):
    mesh = plsc.VectorSubcoreMesh(core_axis_name="c", subcore_axis_name="s")

    @functools.partial(
        pl.kernel,
        out_type=jax.ShapeDtypeStruct((NUM_IDX, EMBED_DIM), table.dtype),
        mesh=mesh,
        scratch_types=[pltpu.SemaphoreType.DMA],
        compiler_params=pltpu.CompilerParams(use_tc_tiling_on_sc=False),
    )
    def gather_kernel(table_hbm, idx_hbm, out_hbm, sem):
        def body(i_vmem, o_vmem):
            copies = [
                pltpu.async_copy(
                    table_hbm.at[i_vmem.at[0, pl.ds(k * WINDOW, WINDOW)]],
                    o_vmem.at[pl.ds(k * WINDOW, WINDOW)],
                    sem,
                )
                for k in range(GATHERS_PER_BODY)
            ]
            for c in copies:
                c.wait()

        pltpu.emit_pipeline(
            body,
            grid=(NUM_IDX // BLOCK,),
            in_specs=[
                pl.BlockSpec((1, BLOCK), index_map=lambda i: (0, i)),
            ],
            out_specs=[
                pl.BlockSpec((BLOCK, EMBED_DIM), index_map=lambda i: (i, 0)),
            ],
            core_axis_name=("c", "s"),
            dimension_semantics=(pltpu.PARALLEL,),
        )(idx_hbm, out_hbm)

    return gather_kernel(table, idx_flat)


def _relayout_tc(x128):
    def body(x_ref, o_ref):
        x = x_ref[...]
        x3 = x.reshape(128, T4, 128)
        for t4 in range(T4):
            y = x3[:, t4, :].T
            o_ref[pl.ds(4 * t4, 4), :, 0, :, :] = y.reshape(4, 4, 8, 128)

    return pl.pallas_call(
        body,
        grid=(BB,),
        in_specs=[pl.BlockSpec((128 * T4, 128), lambda i: (i, 0))],
        out_specs=pl.BlockSpec(
            (MAX_LEN, 4, 1, 8, 128), lambda i: (0, 0, i, 0, 0)
        ),
        out_shape=jax.ShapeDtypeStruct(
            (MAX_LEN, 4, BB, 8, 128), jnp.float32
        ),
    )(x128)


def kernel(inputs, table):
    idx = inputs.reshape(1, NUM_IDX).astype(jnp.int32)
    idx_flat = (idx % GS) * 4 + idx // GS
    tt = jnp.swapaxes(table, 0, 1)
    table4 = _table_relayout_tc(tt).reshape(VP, 32)
    rows = _gather_sc(table4, idx_flat)
    d = _relayout_tc(rows.reshape(NUM_IDX // 4, 128))
    return d.transpose(2, 4, 0, 1, 3).reshape(BATCH, MAX_LEN, EMBED_DIM)
